# baseline (speedup 1.0000x reference)
import functools
import math

import jax
import jax.numpy as jnp
from jax import lax
from jax.experimental import pallas as pl
from jax.experimental.pallas import tpu as pltpu
from jax.experimental.pallas import tpu_sc as plsc

H = 64
NH = 8
HD = 8
NFB = 32
N_PT = 100000
N_PL = 10000
E_PT2PL = 100000
E_PL2PL = 640000

NW = 32
CHUNK = 128
E1P = 102400
E2P = 655360
NSEG_PAD = 10240

SROW = 256
DROW = 128
WROW = 128

_TWO_PI = 2.0 * math.pi


def _lnrow(x, g, b):
    mu = jnp.mean(x, axis=-1, keepdims=True)
    xc = x - mu
    var = jnp.mean(xc * xc, axis=-1, keepdims=True)
    return xc * lax.rsqrt(var + 1e-5) * g + b


def _full(shape):
    return pl.BlockSpec(shape, lambda i: (0,) * len(shape))


def _rows(blk, w):
    return pl.BlockSpec((blk, w), lambda i: (i, 0))


def _sel_mat():
    i64 = lax.broadcasted_iota(jnp.int32, (64, 8), 0)
    i8 = lax.broadcasted_iota(jnp.int32, (64, 8), 1)
    return (i64 // 8 == i8).astype(jnp.float32)


def _sel_mat_t():
    i8 = lax.broadcasted_iota(jnp.int32, (8, 64), 0)
    i64 = lax.broadcasted_iota(jnp.int32, (8, 64), 1)
    return (i64 // 8 == i8).astype(jnp.float32)


def _pt_body(pos, ori, mag, tpt, spt,
             freqs, W1, b1, g1, c1, W2, b2, og, ob, oW, obv,
             embt, embs, lsg, lsb, Wk, Wv, bv, out):
    m = mag[...]
    R = m.shape[0]
    f = m * freqs[...] * _TWO_PI
    trig = jnp.sin(jnp.concatenate([f + 0.5 * math.pi, f], axis=1))
    feat = jnp.concatenate([trig, m], axis=1)
    h = jnp.dot(feat, W1[...]) + b1[...]
    h = jnp.maximum(_lnrow(h, g1[...], c1[...]), 0.0)
    h = jnp.dot(h, W2[...]) + b2[...]
    t = tpt[...]
    s = spt[...]
    oh_t = (t == lax.broadcasted_iota(jnp.int32, (R, 17), 1)).astype(jnp.float32)
    oh_s = (s == lax.broadcasted_iota(jnp.int32, (R, 3), 1)).astype(jnp.float32)
    x = h + jnp.dot(oh_t, embt[...]) + jnp.dot(oh_s, embs[...])
    x = jnp.maximum(_lnrow(x, og[...], ob[...]), 0.0)
    x = jnp.dot(x, oW[...]) + obv[...]
    xs = _lnrow(x, lsg[...], lsb[...])
    k = jnp.dot(xs, Wk[...])
    v = jnp.dot(xs, Wv[...]) + bv[...]
    pad = jnp.zeros((R, 61), jnp.float32)
    pad2 = jnp.zeros((R, 64), jnp.float32)
    out[...] = jnp.concatenate([pos[...], ori[...], pad, k, v, pad2], axis=1)


def _pt_encode(pos_pt, orient_pt, magnitude_pt, type_pt, side_pt, fe, att):
    R = min(4000, N_PT)
    grid = (N_PT // R,)
    v2 = lambda a: a.reshape(1, -1)
    ins = [
        pos_pt, orient_pt.reshape(-1, 1), magnitude_pt.reshape(-1, 1),
        type_pt.reshape(-1, 1).astype(jnp.int32),
        side_pt.reshape(-1, 1).astype(jnp.int32),
        fe['freqs'], fe['W1'].reshape(65, H), v2(fe['b1']),
        v2(fe['ln1_g']), v2(fe['ln1_b']), fe['W2'].reshape(H, H), v2(fe['b2']),
        v2(fe['out_ln_g']), v2(fe['out_ln_b']), fe['out_W'], v2(fe['out_b']),
    ]
    specs = [
        _rows(R, 2), _rows(R, 1), _rows(R, 1), _rows(R, 1), _rows(R, 1),
        _full((1, 32)), _full((65, H)), _full((1, H)),
        _full((1, H)), _full((1, H)), _full((H, H)), _full((1, H)),
        _full((1, H)), _full((1, H)), _full((H, H)), _full((1, H)),
    ]
    ins += [att['embt'], att['embs'], v2(att['ln_src_g']), v2(att['ln_src_b']),
            att['Wq_k'], att['Wq_v'], v2(att['bv'])]
    specs += [_full((17, H)), _full((3, H)), _full((1, H)), _full((1, H)),
              _full((H, H)), _full((H, H)), _full((1, H))]
    return pl.pallas_call(
        _pt_body,
        grid=grid,
        in_specs=specs,
        out_specs=_rows(R, SROW),
        out_shape=jax.ShapeDtypeStruct((N_PT, SROW), jnp.float32),
    )(*ins)


def _pl_body(tpl, ipl, pos, ori, emb4, emb3, og, ob, oW, obv,
             ldg, ldb, Wq, bq, xout, dout):
    t = tpl[...]
    i_ = ipl[...]
    R = t.shape[0]
    oh_t = (t == lax.broadcasted_iota(jnp.int32, (R, 4), 1)).astype(jnp.float32)
    oh_i = (i_ == lax.broadcasted_iota(jnp.int32, (R, 3), 1)).astype(jnp.float32)
    x = jnp.dot(oh_t, emb4[...]) + jnp.dot(oh_i, emb3[...])
    x = jnp.maximum(_lnrow(x, og[...], ob[...]), 0.0)
    x = jnp.dot(x, oW[...]) + obv[...]
    xout[...] = x
    q = jnp.dot(_lnrow(x, ldg[...], ldb[...]), Wq[...]) + bq[...]
    pad = jnp.zeros((R, 61), jnp.float32)
    dout[...] = jnp.concatenate([pos[...], ori[...], pad, q], axis=1)


def _pl_encode(type_pl, is_int_pl, pos_pl, orient_pl, fe, att):
    R = min(2000, N_PL)
    grid = (N_PL // R,)
    v2 = lambda a: a.reshape(1, -1)
    ins = [type_pl.reshape(-1, 1).astype(jnp.int32),
           is_int_pl.reshape(-1, 1).astype(jnp.int32),
           pos_pl, orient_pl.reshape(-1, 1),
           att['emb4'], att['emb3'],
           v2(fe['out_ln_g']), v2(fe['out_ln_b']), fe['out_W'], v2(fe['out_b']),
           v2(att['ln_dst_g']), v2(att['ln_dst_b']), att['Wq'], v2(att['bq'])]
    specs = [_rows(R, 1), _rows(R, 1), _rows(R, 2), _rows(R, 1),
             _full((4, H)), _full((3, H)),
             _full((1, H)), _full((1, H)), _full((H, H)), _full((1, H)),
             _full((1, H)), _full((1, H)), _full((H, H)), _full((1, H))]
    return pl.pallas_call(
        _pl_body,
        grid=grid,
        in_specs=specs,
        out_specs=[_rows(R, H), _rows(R, DROW)],
        out_shape=[jax.ShapeDtypeStruct((N_PL, H), jnp.float32),
                   jax.ShapeDtypeStruct((N_PL, DROW), jnp.float32)],
    )(*ins)


def _lncol(x, g, b):
    F = x.shape[0]
    ones = jnp.full((1, F), 1.0 / F, jnp.float32)
    mu = jnp.dot(ones, x)
    ctr = x - mu
    var = jnp.dot(ones, ctr * ctr)
    return ctr * lax.rsqrt(var + 1e-5) * g + b


def _edge_body(ne_real, BE, split_geom, *refs):
    if split_geom:
        (srow_r, drow_r, tv_r, geom_r,
         freqs_r, W1_r, b1_r, g1_r, c1_r, W2_r, b2_r,
         og_r, ob_r, oW_r, obv_r, emb5_r,
         lrg_r, lrb_r, Wkr_r, Wvr_r, bvr_r, out_r) = refs
    else:
        (srow_r, drow_r, tv_r,
         freqs_r, W1_r, b1_r, g1_r, c1_r, W2_r, b2_r,
         og_r, ob_r, oW_r, obv_r, emb5_r,
         lrg_r, lrb_r, Wkr_r, Wvr_r, bvr_r, out_r) = refs
    srow = srow_r[...]
    drow = drow_r[...]
    if split_geom:
        g17 = jnp.concatenate([geom_r[...], drow[:, 0:8], tv_r[...]], axis=1)
        gT = jnp.transpose(g17)
        sx, sy, so = gT[0:1], gT[1:2], gT[2:3]
        dx, dy, do_ = gT[8:9], gT[9:10], gT[10:11]
        tT = gT[16:17]
    else:
        g17 = jnp.concatenate([srow[:, 0:8], drow[:, 0:8], tv_r[...]], axis=1)
        gT = jnp.transpose(g17)
        sx, sy, so = gT[0:1], gT[1:2], gT[2:3]
        dx, dy, do_ = gT[8:9], gT[9:10], gT[10:11]
        tT = gT[16:17]
    rpx = sx - dx
    rpy = sy - dy
    ro = jnp.mod(so - do_ + math.pi, _TWO_PI) - math.pi
    ang = jnp.mod(jnp.arctan2(rpy, rpx) - do_ + math.pi, _TWO_PI) - math.pi
    dist = jnp.sqrt(rpx * rpx + rpy * rpy)
    contT = jnp.concatenate([dist, ang, ro], axis=0)
    i96 = lax.broadcasted_iota(jnp.int32, (96, 3), 0)
    i3 = lax.broadcasted_iota(jnp.int32, (96, 3), 1)
    E3T = (i96 // 32 == i3).astype(jnp.float32)
    fT = jnp.dot(E3T, contT) * freqs_r[...] * _TWO_PI
    featT = jnp.concatenate([jnp.cos(fT), jnp.sin(fT), contT], axis=0)
    HT = jnp.dot(W1_r[...], featT) + b1_r[...]
    ig3 = lax.broadcasted_iota(jnp.int32, (3, 192), 0)
    ig = lax.broadcasted_iota(jnp.int32, (3, 192), 1)
    GdT = (ig // 64 == ig3).astype(jnp.float32) * (1.0 / 64.0)
    jg = lax.broadcasted_iota(jnp.int32, (192, 3), 0)
    jg3 = lax.broadcasted_iota(jnp.int32, (192, 3), 1)
    GtT = (jg // 64 == jg3).astype(jnp.float32)
    mu3 = jnp.dot(GdT, HT)
    ctr = HT - jnp.dot(GtT, mu3)
    var3 = jnp.dot(GdT, ctr * ctr)
    sc3 = lax.rsqrt(var3 + 1e-5)
    Hn = ctr * jnp.dot(GtT, sc3) * g1_r[...] + c1_r[...]
    Hr = jnp.maximum(Hn, 0.0)
    xT = jnp.dot(W2_r[...], Hr) + b2_r[...]
    i5 = lax.broadcasted_iota(jnp.int32, (5, BE), 0).astype(jnp.float32)
    ohT = (i5 == tT).astype(jnp.float32)
    xT = xT + jnp.dot(emb5_r[...], ohT)
    xT = jnp.maximum(_lncol(xT, og_r[...], ob_r[...]), 0.0)
    xT = jnp.dot(oW_r[...], xT) + obv_r[...]
    rnT = _lncol(xT, lrg_r[...], lrb_r[...])
    krT = jnp.dot(Wkr_r[...], rnT)
    vrT = jnp.dot(Wvr_r[...], rnT) + bvr_r[...]
    if split_geom:
        ksT = jnp.transpose(srow[:, 0:64])
        vsT = jnp.transpose(srow[:, 64:128])
    else:
        ksT = jnp.transpose(srow[:, 64:128])
        vsT = jnp.transpose(srow[:, 128:192])
    qdT = jnp.transpose(drow[:, 64:128])
    kjT = ksT + krT
    vjT = vsT + vrT
    qkT = qdT * kjT
    ST = _sel_mat_t()
    S = _sel_mat()
    simT = jnp.dot(ST, qkT) * (1.0 / math.sqrt(float(HD)))
    eid = pl.program_id(0) * BE + lax.broadcasted_iota(jnp.int32, (NH, BE), 1)
    esimT = jnp.exp(simT)
    esimT = jnp.where(eid < ne_real, esimT, 0.0)
    wvT = vjT * jnp.dot(S, esimT)
    esim = jnp.transpose(esimT)
    wv = jnp.transpose(wvT)
    pad = jnp.zeros((BE, 56), jnp.float32)
    out_r[...] = jnp.concatenate([esim, pad, wv], axis=1)


def _w1_blockdiag(W1):
    Wp = jnp.zeros((195, 3 * H), jnp.float32)
    for d in range(3):
        Wp = Wp.at[32 * d:32 * d + 32, H * d:H * d + H].set(W1[d, :32])
        Wp = Wp.at[96 + 32 * d:96 + 32 * d + 32, H * d:H * d + H].set(W1[d, 32:64])
        Wp = Wp.at[192 + d, H * d:H * d + H].set(W1[d, 64])
    return Wp


def _edge_call(ne_pad, ne_real, srows, drows, tvec, fe, emb5, att, geomT=None):
    BE = 4096
    grid = (ne_pad // BE,)
    vc = lambda a: a.reshape(-1, 1)
    split_geom = geomT is not None
    srw = 128 if split_geom else SROW
    ins = [srows, drows, tvec.astype(jnp.float32).reshape(-1, 1),
           fe['freqs'].reshape(96, 1), _w1_blockdiag(fe['W1']).T,
           vc(fe['b1']), vc(fe['ln1_g']), vc(fe['ln1_b']),
           fe['W2'].reshape(3 * H, H).T, vc(jnp.sum(fe['b2'], axis=0)),
           vc(fe['out_ln_g']), vc(fe['out_ln_b']), fe['out_W'].T,
           vc(fe['out_b']),
           emb5.T,
           vc(att['ln_r_g']), vc(att['ln_r_b']),
           att['Wkr'].T, att['Wvr'].T, vc(att['bvr'])]
    specs = [_rows(BE, srw), _rows(BE, DROW), _rows(BE, 1),
             _full((96, 1)), _full((3 * H, 195)),
             _full((3 * H, 1)), _full((3 * H, 1)), _full((3 * H, 1)),
             _full((H, 3 * H)), _full((H, 1)),
             _full((H, 1)), _full((H, 1)), _full((H, H)),
             _full((H, 1)),
             _full((H, 5)),
             _full((H, 1)), _full((H, 1)),
             _full((H, H)), _full((H, H)), _full((H, 1))]
    if split_geom:
        ins.insert(3, geomT)
        specs.insert(3, _rows(BE, 8))
    return pl.pallas_call(
        functools.partial(_edge_body, ne_real, BE, split_geom),
        grid=grid,
        in_specs=specs,
        out_specs=_rows(BE, WROW),
        out_shape=jax.ShapeDtypeStruct((ne_pad, WROW), jnp.float32),
    )(*ins)


def _node_body(emit_next, ldg, ldb, Wga, Wgx, bg, Ws, bs, Wo, bo,
               lfg, lfb, fW1, fb1, fW2, fb2, *rest):
    if emit_next:
        (P0, P1, xdst, pos, ori,
         l2sg, l2sb, l2dg, l2db, Wk2, Wv2, bv2, Wq2, bq2,
         xout, stab, dtab) = rest
    else:
        P0, P1, xdst, xout = rest
    acc = P0[...] + P1[...]
    s = acc[:, 0:NH]
    Pv = acc[:, 64:128]
    ST = _sel_mat_t()
    sinv = 1.0 / (s + 1e-16)
    agg = Pv * jnp.dot(sinv, ST)
    xd = _lnrow(xdst[...], ldg[...], ldb[...])
    gpre = jnp.dot(agg, Wga[...]) + jnp.dot(xd, Wgx[...]) + bg[...]
    g = 1.0 / (1.0 + jnp.exp(-gpre))
    upd = agg + g * ((jnp.dot(xd, Ws[...]) + bs[...]) - agg)
    x = xdst[...] + jnp.dot(upd, Wo[...]) + bo[...]
    xf = _lnrow(x, lfg[...], lfb[...])
    xnew = x + jnp.dot(jnp.maximum(jnp.dot(xf, fW1[...]) + fb1[...], 0.0),
                       fW2[...]) + fb2[...]
    xout[...] = xnew
    if emit_next:
        R = xnew.shape[0]
        xs2 = _lnrow(xnew, l2sg[...], l2sb[...])
        xd2 = _lnrow(xnew, l2dg[...], l2db[...])
        k2 = jnp.dot(xs2, Wk2[...])
        v2_ = jnp.dot(xs2, Wv2[...]) + bv2[...]
        q2 = jnp.dot(xd2, Wq2[...]) + bq2[...]
        pad = jnp.zeros((R, 61), jnp.float32)
        stab[...] = jnp.concatenate([k2, v2_], axis=1)
        dtab[...] = jnp.concatenate([pos[...], ori[...], pad, q2], axis=1)


def _node_call(P, xdst, att, nxt=None):
    R = min(2000, N_PL)
    grid = (N_PL // R,)
    v2 = lambda a: a.reshape(1, -1)
    ins = [v2(att['ln_dst_g']), v2(att['ln_dst_b']),
           att['Wg'][:H, :], att['Wg'][H:, :], v2(att['bg']),
           att['Ws'], v2(att['bs']), att['Wo'], v2(att['bo']),
           v2(att['ln_ff_g']), v2(att['ln_ff_b']),
           att['ffW1'], v2(att['ffb1']), att['ffW2'], v2(att['ffb2']),
           P[:N_PL], P[NSEG_PAD:NSEG_PAD + N_PL], xdst]
    specs = [_full((1, H)), _full((1, H)),
             _full((H, H)), _full((H, H)), _full((1, H)),
             _full((H, H)), _full((1, H)), _full((H, H)), _full((1, H)),
             _full((1, H)), _full((1, H)),
             _full((H, 4 * H)), _full((1, 4 * H)), _full((4 * H, H)), _full((1, H)),
             _rows(R, WROW), _rows(R, WROW), _rows(R, H)]
    emit_next = nxt is not None
    if emit_next:
        pos, ori, att2 = nxt
        ins += [pos, ori.reshape(-1, 1),
                v2(att2['ln_src_g']), v2(att2['ln_src_b']),
                v2(att2['ln_dst_g']), v2(att2['ln_dst_b']),
                att2['Wk'], att2['Wv'], v2(att2['bv']),
                att2['Wq'], v2(att2['bq'])]
        specs += [_rows(R, 2), _rows(R, 1),
                  _full((1, H)), _full((1, H)), _full((1, H)), _full((1, H)),
                  _full((H, H)), _full((H, H)), _full((1, H)),
                  _full((H, H)), _full((1, H))]
        out_specs = [_rows(R, H), _rows(R, 2 * H), _rows(R, DROW)]
        out_shape = [jax.ShapeDtypeStruct((N_PL, H), jnp.float32),
                     jax.ShapeDtypeStruct((N_PL, 2 * H), jnp.float32),
                     jax.ShapeDtypeStruct((N_PL, DROW), jnp.float32)]
    else:
        out_specs = _rows(R, H)
        out_shape = jax.ShapeDtypeStruct((N_PL, H), jnp.float32)
    return pl.pallas_call(
        functools.partial(_node_body, emit_next),
        grid=grid,
        in_specs=specs,
        out_specs=out_specs,
        out_shape=out_shape,
    )(*ins)


def _sc_gather2(E, srw):
    per_w = E // NW
    n_chunks = per_w // CHUNK
    mesh = plsc.VectorSubcoreMesh(core_axis_name="c", subcore_axis_name="s")

    @functools.partial(
        pl.kernel,
        out_type=[jax.ShapeDtypeStruct((E, srw), jnp.float32),
                  jax.ShapeDtypeStruct((E, DROW), jnp.float32)],
        mesh=mesh,
        scratch_types=[
            pltpu.VMEM((CHUNK,), jnp.int32),
            pltpu.VMEM((CHUNK,), jnp.int32),
            pltpu.VMEM((CHUNK, srw), jnp.float32),
            pltpu.VMEM((CHUNK, DROW), jnp.float32),
            pltpu.SemaphoreType.DMA,
            pltpu.SemaphoreType.DMA,
        ],
    )
    def k(stab, sidx, dtab, didx, outs, outd, siv, div, srows, drows, sem1, sem2):
        c = lax.axis_index("c")
        s = lax.axis_index("s")
        wid = s * 2 + c
        base = wid * per_w

        def body(i, carry):
            off = base + i * CHUNK
            pltpu.sync_copy(sidx.at[pl.ds(off, CHUNK)], siv)
            pltpu.sync_copy(didx.at[pl.ds(off, CHUNK)], div)
            cp1 = pltpu.async_copy(stab.at[siv], srows, sem1)
            cp2 = pltpu.async_copy(dtab.at[div], drows, sem2)
            cp1.wait()
            cp2.wait()
            pltpu.sync_copy(srows, outs.at[pl.ds(off, CHUNK)])
            pltpu.sync_copy(drows, outd.at[pl.ds(off, CHUNK)])
            return carry

        lax.fori_loop(0, n_chunks, body, 0)

    return k


def _sc_geom_gather(E):
    per_w = E // NW
    n_chunks = per_w // CHUNK
    rows_per_tile = NSEG_PAD // 16
    mesh = plsc.VectorSubcoreMesh(core_axis_name="c", subcore_axis_name="s")

    @functools.partial(
        pl.kernel,
        out_type=jax.ShapeDtypeStruct((E, 8), jnp.float32),
        mesh=mesh,
        scratch_types=[
            pltpu.VMEM((CHUNK,), jnp.int32),
            pltpu.VMEM((CHUNK, 8), jnp.float32),
            pltpu.VMEM_SHARED((NSEG_PAD, 8), jnp.float32),
            pltpu.SemaphoreType.DMA,
        ],
    )
    def k(gtab_hbm, sidx_hbm, out_hbm, idx_v, rows_v, gtab_sh, sem):
        c = lax.axis_index("c")
        s = lax.axis_index("s")
        wid = s * 2 + c
        base = wid * per_w
        r0 = s * rows_per_tile
        pltpu.sync_copy(gtab_hbm.at[pl.ds(r0, rows_per_tile)],
                        gtab_sh.at[pl.ds(r0, rows_per_tile)])
        plsc.subcore_barrier()

        def body(i, carry):
            off = base + i * CHUNK
            pltpu.sync_copy(sidx_hbm.at[pl.ds(off, CHUNK)], idx_v)
            pltpu.async_copy(gtab_sh.at[idx_v], rows_v, sem).wait()
            pltpu.sync_copy(rows_v, out_hbm.at[pl.ds(off, CHUNK)])
            return carry

        lax.fori_loop(0, n_chunks, body, 0)

    return k


def _sc_scatter(E):
    per_c = E // 2
    per_w = per_c // 16
    n_chunks = per_w // CHUNK
    rows_per_tile = NSEG_PAD // 16
    mesh = plsc.VectorSubcoreMesh(core_axis_name="c", subcore_axis_name="s")

    @functools.partial(
        pl.kernel,
        out_type=jax.ShapeDtypeStruct((2 * NSEG_PAD, WROW), jnp.float32),
        mesh=mesh,
        scratch_types=[
            pltpu.VMEM((CHUNK,), jnp.int32),
            pltpu.VMEM((CHUNK, WROW), jnp.float32),
            pltpu.VMEM_SHARED((NSEG_PAD, WROW), jnp.float32),
        ],
    )
    def k(w_hbm, idx_hbm, zeros_hbm, out_hbm, idx_v, w_v, accum):
        c = lax.axis_index("c")
        s = lax.axis_index("s")
        r0 = s * rows_per_tile
        pltpu.sync_copy(zeros_hbm.at[pl.ds(r0, rows_per_tile)],
                        accum.at[pl.ds(r0, rows_per_tile)])
        plsc.subcore_barrier()
        base = c * per_c + s * per_w

        def body(i, carry):
            off = base + i * CHUNK
            pltpu.sync_copy(idx_hbm.at[pl.ds(off, CHUNK)], idx_v)
            pltpu.sync_copy(w_hbm.at[pl.ds(off, CHUNK)], w_v)
            pltpu.sync_copy(w_v, accum.at[idx_v], add=True)
            return carry

        lax.fori_loop(0, n_chunks, body, 0)
        plsc.subcore_barrier()
        pltpu.sync_copy(accum.at[pl.ds(r0, rows_per_tile)],
                        out_hbm.at[pl.ds(c * NSEG_PAD + r0, rows_per_tile)])

    return k


def _pad_i32(a, n):
    return jnp.pad(a.astype(jnp.int32), (0, n - a.shape[0]))


def kernel(params, pos_pt, orient_pt, magnitude_pt, pos_pl, orient_pl,
           type_pt, side_pt, type_pl, is_intersection_pl,
           edge_index_pt2pl, edge_index_pl2pl, type_pl2pl):
    p = params
    att1 = dict(p['pt2pl_layers'][0])
    att2 = dict(p['pl2pl_layers'][0])
    att1['embt'] = p['emb_type_pt']
    att1['embs'] = p['emb_side_pt']
    att1['emb4'] = p['emb_type_pl']
    att1['emb3'] = p['emb_int_pl']
    att1['Wq_k'] = att1['Wk']
    att1['Wq_v'] = att1['Wv']

    stab_pt = _pt_encode(pos_pt, orient_pt, magnitude_pt, type_pt, side_pt,
                         p['fe_x_pt'], att1)
    x_pl0, dtab0 = _pl_encode(type_pl, is_intersection_pl, pos_pl, orient_pl,
                              p['fe_x_pl'], att1)

    s1 = _pad_i32(edge_index_pt2pl[0], E1P)
    d1 = _pad_i32(edge_index_pt2pl[1], E1P)
    srows1, drows1 = _sc_gather2(E1P, SROW)(stab_pt, s1, dtab0, d1)

    t1 = jnp.zeros((E1P,), jnp.int32)
    emb5_zero = jnp.zeros((5, H), jnp.float32)
    w1 = _edge_call(E1P, E_PT2PL, srows1, drows1, t1,
                    p['fe_r_pt2pl'], emb5_zero, att1)

    zeros_acc = jnp.zeros((NSEG_PAD, WROW), jnp.float32)
    P1 = _sc_scatter(E1P)(w1, d1, zeros_acc)

    x_pl1, stab_pl, dtab1 = _node_call(P1, x_pl0, att1,
                                       nxt=(pos_pl, orient_pl, att2))

    s2 = _pad_i32(edge_index_pl2pl[0], E2P)
    d2 = _pad_i32(edge_index_pl2pl[1], E2P)
    geomtab = jnp.zeros((NSEG_PAD, 8), jnp.float32)
    geomtab = geomtab.at[:N_PL, 0:2].set(pos_pl)
    geomtab = geomtab.at[:N_PL, 2].set(orient_pl)
    geomT2 = _sc_geom_gather(E2P)(geomtab, s2)
    srows2, drows2 = _sc_gather2(E2P, 128)(stab_pl, s2, dtab1, d2)

    t2 = _pad_i32(type_pl2pl, E2P)
    w2 = _edge_call(E2P, E_PL2PL, srows2, drows2, t2,
                    p['fe_r_pl2pl'], p['emb_type_pl2pl'], att2, geomT=geomT2)

    P2 = _sc_scatter(E2P)(w2, d2, zeros_acc)

    return _node_call(P2, x_pl1, att2)


# speedup vs baseline: 1.1663x; 1.1663x over previous
import functools
import math

import jax
import jax.numpy as jnp
from jax import lax
from jax.experimental import pallas as pl
from jax.experimental.pallas import tpu as pltpu
from jax.experimental.pallas import tpu_sc as plsc

H = 64
NH = 8
HD = 8
NFB = 32
N_PT = 100000
N_PL = 10000
E_PT2PL = 100000
E_PL2PL = 640000

NW = 32
CHUNK = 128
E1P = 102400
E2P = 643072
NSEG_PAD = 10240

SROW = 256
DROW = 128
WROW = 128

_TWO_PI = 2.0 * math.pi


def _lnrow(x, g, b):
    mu = jnp.mean(x, axis=-1, keepdims=True)
    xc = x - mu
    var = jnp.mean(xc * xc, axis=-1, keepdims=True)
    return xc * lax.rsqrt(var + 1e-5) * g + b


def _full(shape):
    return pl.BlockSpec(shape, lambda i: (0,) * len(shape))


def _rows(blk, w):
    return pl.BlockSpec((blk, w), lambda i: (i, 0))


def _sel_mat():
    i64 = lax.broadcasted_iota(jnp.int32, (64, 8), 0)
    i8 = lax.broadcasted_iota(jnp.int32, (64, 8), 1)
    return (i64 // 8 == i8).astype(jnp.float32)


def _sel_mat_t():
    i8 = lax.broadcasted_iota(jnp.int32, (8, 64), 0)
    i64 = lax.broadcasted_iota(jnp.int32, (8, 64), 1)
    return (i64 // 8 == i8).astype(jnp.float32)


def _pt_body(pos, ori, mag, tpt, spt,
             freqs, W1, b1, g1, c1, W2, b2, og, ob, oW, obv,
             embt, embs, lsg, lsb, Wk, Wv, bv, out):
    m = mag[...]
    R = m.shape[0]
    f = m * freqs[...] * _TWO_PI
    trig = jnp.sin(jnp.concatenate([f + 0.5 * math.pi, f], axis=1))
    feat = jnp.concatenate([trig, m], axis=1)
    h = jnp.dot(feat, W1[...]) + b1[...]
    h = jnp.maximum(_lnrow(h, g1[...], c1[...]), 0.0)
    h = jnp.dot(h, W2[...]) + b2[...]
    t = tpt[...]
    s = spt[...]
    oh_t = (t == lax.broadcasted_iota(jnp.int32, (R, 17), 1)).astype(jnp.float32)
    oh_s = (s == lax.broadcasted_iota(jnp.int32, (R, 3), 1)).astype(jnp.float32)
    x = h + jnp.dot(oh_t, embt[...]) + jnp.dot(oh_s, embs[...])
    x = jnp.maximum(_lnrow(x, og[...], ob[...]), 0.0)
    x = jnp.dot(x, oW[...]) + obv[...]
    xs = _lnrow(x, lsg[...], lsb[...])
    k = jnp.dot(xs, Wk[...])
    v = jnp.dot(xs, Wv[...]) + bv[...]
    pad = jnp.zeros((R, 61), jnp.float32)
    pad2 = jnp.zeros((R, 64), jnp.float32)
    out[...] = jnp.concatenate([pos[...], ori[...], pad, k, v, pad2], axis=1)


def _pt_encode(pos_pt, orient_pt, magnitude_pt, type_pt, side_pt, fe, att):
    R = min(4000, N_PT)
    grid = (N_PT // R,)
    v2 = lambda a: a.reshape(1, -1)
    ins = [
        pos_pt, orient_pt.reshape(-1, 1), magnitude_pt.reshape(-1, 1),
        type_pt.reshape(-1, 1).astype(jnp.int32),
        side_pt.reshape(-1, 1).astype(jnp.int32),
        fe['freqs'], fe['W1'].reshape(65, H), v2(fe['b1']),
        v2(fe['ln1_g']), v2(fe['ln1_b']), fe['W2'].reshape(H, H), v2(fe['b2']),
        v2(fe['out_ln_g']), v2(fe['out_ln_b']), fe['out_W'], v2(fe['out_b']),
    ]
    specs = [
        _rows(R, 2), _rows(R, 1), _rows(R, 1), _rows(R, 1), _rows(R, 1),
        _full((1, 32)), _full((65, H)), _full((1, H)),
        _full((1, H)), _full((1, H)), _full((H, H)), _full((1, H)),
        _full((1, H)), _full((1, H)), _full((H, H)), _full((1, H)),
    ]
    ins += [att['embt'], att['embs'], v2(att['ln_src_g']), v2(att['ln_src_b']),
            att['Wq_k'], att['Wq_v'], v2(att['bv'])]
    specs += [_full((17, H)), _full((3, H)), _full((1, H)), _full((1, H)),
              _full((H, H)), _full((H, H)), _full((1, H))]
    return pl.pallas_call(
        _pt_body,
        grid=grid,
        in_specs=specs,
        out_specs=_rows(R, SROW),
        out_shape=jax.ShapeDtypeStruct((N_PT, SROW), jnp.float32),
    )(*ins)


def _pl_body(tpl, ipl, pos, ori, emb4, emb3, og, ob, oW, obv,
             ldg, ldb, Wq, bq, xout, dout):
    t = tpl[...]
    i_ = ipl[...]
    R = t.shape[0]
    oh_t = (t == lax.broadcasted_iota(jnp.int32, (R, 4), 1)).astype(jnp.float32)
    oh_i = (i_ == lax.broadcasted_iota(jnp.int32, (R, 3), 1)).astype(jnp.float32)
    x = jnp.dot(oh_t, emb4[...]) + jnp.dot(oh_i, emb3[...])
    x = jnp.maximum(_lnrow(x, og[...], ob[...]), 0.0)
    x = jnp.dot(x, oW[...]) + obv[...]
    xout[...] = x
    q = jnp.dot(_lnrow(x, ldg[...], ldb[...]), Wq[...]) + bq[...]
    pad = jnp.zeros((R, 61), jnp.float32)
    dout[...] = jnp.concatenate([pos[...], ori[...], pad, q], axis=1)


def _pl_encode(type_pl, is_int_pl, pos_pl, orient_pl, fe, att):
    R = min(2000, N_PL)
    grid = (N_PL // R,)
    v2 = lambda a: a.reshape(1, -1)
    ins = [type_pl.reshape(-1, 1).astype(jnp.int32),
           is_int_pl.reshape(-1, 1).astype(jnp.int32),
           pos_pl, orient_pl.reshape(-1, 1),
           att['emb4'], att['emb3'],
           v2(fe['out_ln_g']), v2(fe['out_ln_b']), fe['out_W'], v2(fe['out_b']),
           v2(att['ln_dst_g']), v2(att['ln_dst_b']), att['Wq'], v2(att['bq'])]
    specs = [_rows(R, 1), _rows(R, 1), _rows(R, 2), _rows(R, 1),
             _full((4, H)), _full((3, H)),
             _full((1, H)), _full((1, H)), _full((H, H)), _full((1, H)),
             _full((1, H)), _full((1, H)), _full((H, H)), _full((1, H))]
    return pl.pallas_call(
        _pl_body,
        grid=grid,
        in_specs=specs,
        out_specs=[_rows(R, H), _rows(R, DROW)],
        out_shape=[jax.ShapeDtypeStruct((N_PL, H), jnp.float32),
                   jax.ShapeDtypeStruct((N_PL, DROW), jnp.float32)],
    )(*ins)


def _lncol(x, g, b):
    F = x.shape[0]
    ones = jnp.full((1, F), 1.0 / F, jnp.float32)
    mu = jnp.dot(ones, x)
    ctr = x - mu
    var = jnp.dot(ones, ctr * ctr)
    return ctr * lax.rsqrt(var + 1e-5) * g + b


def _edge_body(ne_real, BE, srow_r, drow_r, tv_r,
               freqs_r, W1_r, b1_r, g1_r, c1_r, W2_r, b2_r,
               og_r, ob_r, oW_r, obv_r, emb5_r,
               lrg_r, lrb_r, Wkr_r, Wvr_r, bvr_r, out_r):
    srow = srow_r[...]
    drow = drow_r[...]
    g17 = jnp.concatenate([srow[:, 0:8], drow[:, 0:8], tv_r[...]], axis=1)
    gT = jnp.transpose(g17)
    sx, sy, so = gT[0:1], gT[1:2], gT[2:3]
    dx, dy, do_ = gT[8:9], gT[9:10], gT[10:11]
    tT = gT[16:17]
    rpx = sx - dx
    rpy = sy - dy
    ro = jnp.mod(so - do_ + math.pi, _TWO_PI) - math.pi
    ang = jnp.mod(jnp.arctan2(rpy, rpx) - do_ + math.pi, _TWO_PI) - math.pi
    dist = jnp.sqrt(rpx * rpx + rpy * rpy)
    contT = jnp.concatenate([dist, ang, ro], axis=0)
    i96 = lax.broadcasted_iota(jnp.int32, (96, 3), 0)
    i3 = lax.broadcasted_iota(jnp.int32, (96, 3), 1)
    E3T = (i96 // 32 == i3).astype(jnp.float32)
    fT = jnp.dot(E3T, contT) * freqs_r[...] * _TWO_PI
    featT = jnp.concatenate([jnp.cos(fT), jnp.sin(fT), contT], axis=0)
    HT = jnp.dot(W1_r[...], featT) + b1_r[...]
    ig3 = lax.broadcasted_iota(jnp.int32, (3, 192), 0)
    ig = lax.broadcasted_iota(jnp.int32, (3, 192), 1)
    GdT = (ig // 64 == ig3).astype(jnp.float32) * (1.0 / 64.0)
    jg = lax.broadcasted_iota(jnp.int32, (192, 3), 0)
    jg3 = lax.broadcasted_iota(jnp.int32, (192, 3), 1)
    GtT = (jg // 64 == jg3).astype(jnp.float32)
    mu3 = jnp.dot(GdT, HT)
    ctr = HT - jnp.dot(GtT, mu3)
    var3 = jnp.dot(GdT, ctr * ctr)
    sc3 = lax.rsqrt(var3 + 1e-5)
    Hn = ctr * jnp.dot(GtT, sc3) * g1_r[...] + c1_r[...]
    Hr = jnp.maximum(Hn, 0.0)
    xT = jnp.dot(W2_r[...], Hr) + b2_r[...]
    i5 = lax.broadcasted_iota(jnp.int32, (5, BE), 0).astype(jnp.float32)
    ohT = (i5 == tT).astype(jnp.float32)
    xT = xT + jnp.dot(emb5_r[...], ohT)
    xT = jnp.maximum(_lncol(xT, og_r[...], ob_r[...]), 0.0)
    xT = jnp.dot(oW_r[...], xT) + obv_r[...]
    rnT = _lncol(xT, lrg_r[...], lrb_r[...])
    krT = jnp.dot(Wkr_r[...], rnT)
    vrT = jnp.dot(Wvr_r[...], rnT) + bvr_r[...]
    ksT = jnp.transpose(srow[:, 64:128])
    vsT = jnp.transpose(srow[:, 128:192])
    qdT = jnp.transpose(drow[:, 64:128])
    kjT = ksT + krT
    vjT = vsT + vrT
    qkT = qdT * kjT
    ST = _sel_mat_t()
    S = _sel_mat()
    simT = jnp.dot(ST, qkT) * (1.0 / math.sqrt(float(HD)))
    eid = pl.program_id(0) * BE + lax.broadcasted_iota(jnp.int32, (NH, BE), 1)
    esimT = jnp.exp(simT)
    esimT = jnp.where(eid < ne_real, esimT, 0.0)
    wvT = vjT * jnp.dot(S, esimT)
    esim = jnp.transpose(esimT)
    wv = jnp.transpose(wvT)
    pad = jnp.zeros((BE, 56), jnp.float32)
    out_r[...] = jnp.concatenate([esim, pad, wv], axis=1)


def _w1_blockdiag(W1):
    Wp = jnp.zeros((195, 3 * H), jnp.float32)
    for d in range(3):
        Wp = Wp.at[32 * d:32 * d + 32, H * d:H * d + H].set(W1[d, :32])
        Wp = Wp.at[96 + 32 * d:96 + 32 * d + 32, H * d:H * d + H].set(W1[d, 32:64])
        Wp = Wp.at[192 + d, H * d:H * d + H].set(W1[d, 64])
    return Wp


def _edge_call(ne_pad, ne_real, srows, drows, tvec, fe, emb5, att):
    BE = 4096
    grid = (ne_pad // BE,)
    vc = lambda a: a.reshape(-1, 1)
    ins = [srows, drows, tvec.astype(jnp.float32).reshape(-1, 1),
           fe['freqs'].reshape(96, 1), _w1_blockdiag(fe['W1']).T,
           vc(fe['b1']), vc(fe['ln1_g']), vc(fe['ln1_b']),
           fe['W2'].reshape(3 * H, H).T, vc(jnp.sum(fe['b2'], axis=0)),
           vc(fe['out_ln_g']), vc(fe['out_ln_b']), fe['out_W'].T,
           vc(fe['out_b']),
           emb5.T,
           vc(att['ln_r_g']), vc(att['ln_r_b']),
           att['Wkr'].T, att['Wvr'].T, vc(att['bvr'])]
    specs = [_rows(BE, SROW), _rows(BE, DROW), _rows(BE, 1),
             _full((96, 1)), _full((3 * H, 195)),
             _full((3 * H, 1)), _full((3 * H, 1)), _full((3 * H, 1)),
             _full((H, 3 * H)), _full((H, 1)),
             _full((H, 1)), _full((H, 1)), _full((H, H)),
             _full((H, 1)),
             _full((H, 5)),
             _full((H, 1)), _full((H, 1)),
             _full((H, H)), _full((H, H)), _full((H, 1))]
    return pl.pallas_call(
        functools.partial(_edge_body, ne_real, BE),
        grid=grid,
        in_specs=specs,
        out_specs=_rows(BE, WROW),
        out_shape=jax.ShapeDtypeStruct((ne_pad, WROW), jnp.float32),
    )(*ins)


def _node_body(emit_next, ldg, ldb, Wga, Wgx, bg, Ws, bs, Wo, bo,
               lfg, lfb, fW1, fb1, fW2, fb2, *rest):
    if emit_next:
        (P0, P1, xdst, pos, ori,
         l2sg, l2sb, l2dg, l2db, Wk2, Wv2, bv2, Wq2, bq2,
         xout, stab, dtab) = rest
    else:
        P0, P1, xdst, xout = rest
    acc = P0[...] + P1[...]
    s = acc[:, 0:NH]
    Pv = acc[:, 64:128]
    ST = _sel_mat_t()
    sinv = 1.0 / (s + 1e-16)
    agg = Pv * jnp.dot(sinv, ST)
    xd = _lnrow(xdst[...], ldg[...], ldb[...])
    gpre = jnp.dot(agg, Wga[...]) + jnp.dot(xd, Wgx[...]) + bg[...]
    g = 1.0 / (1.0 + jnp.exp(-gpre))
    upd = agg + g * ((jnp.dot(xd, Ws[...]) + bs[...]) - agg)
    x = xdst[...] + jnp.dot(upd, Wo[...]) + bo[...]
    xf = _lnrow(x, lfg[...], lfb[...])
    xnew = x + jnp.dot(jnp.maximum(jnp.dot(xf, fW1[...]) + fb1[...], 0.0),
                       fW2[...]) + fb2[...]
    xout[...] = xnew
    if emit_next:
        R = xnew.shape[0]
        xs2 = _lnrow(xnew, l2sg[...], l2sb[...])
        xd2 = _lnrow(xnew, l2dg[...], l2db[...])
        k2 = jnp.dot(xs2, Wk2[...])
        v2_ = jnp.dot(xs2, Wv2[...]) + bv2[...]
        q2 = jnp.dot(xd2, Wq2[...]) + bq2[...]
        pad = jnp.zeros((R, 61), jnp.float32)
        pad2 = jnp.zeros((R, 64), jnp.float32)
        stab[...] = jnp.concatenate([pos[...], ori[...], pad, k2, v2_, pad2],
                                    axis=1)
        dtab[...] = jnp.concatenate([pos[...], ori[...], pad, q2], axis=1)


def _node_call(P, xdst, att, nxt=None):
    R = min(2000, N_PL)
    grid = (N_PL // R,)
    v2 = lambda a: a.reshape(1, -1)
    ins = [v2(att['ln_dst_g']), v2(att['ln_dst_b']),
           att['Wg'][:H, :], att['Wg'][H:, :], v2(att['bg']),
           att['Ws'], v2(att['bs']), att['Wo'], v2(att['bo']),
           v2(att['ln_ff_g']), v2(att['ln_ff_b']),
           att['ffW1'], v2(att['ffb1']), att['ffW2'], v2(att['ffb2']),
           P[:N_PL], P[NSEG_PAD:NSEG_PAD + N_PL], xdst]
    specs = [_full((1, H)), _full((1, H)),
             _full((H, H)), _full((H, H)), _full((1, H)),
             _full((H, H)), _full((1, H)), _full((H, H)), _full((1, H)),
             _full((1, H)), _full((1, H)),
             _full((H, 4 * H)), _full((1, 4 * H)), _full((4 * H, H)), _full((1, H)),
             _rows(R, WROW), _rows(R, WROW), _rows(R, H)]
    emit_next = nxt is not None
    if emit_next:
        pos, ori, att2 = nxt
        ins += [pos, ori.reshape(-1, 1),
                v2(att2['ln_src_g']), v2(att2['ln_src_b']),
                v2(att2['ln_dst_g']), v2(att2['ln_dst_b']),
                att2['Wk'], att2['Wv'], v2(att2['bv']),
                att2['Wq'], v2(att2['bq'])]
        specs += [_rows(R, 2), _rows(R, 1),
                  _full((1, H)), _full((1, H)), _full((1, H)), _full((1, H)),
                  _full((H, H)), _full((H, H)), _full((1, H)),
                  _full((H, H)), _full((1, H))]
        out_specs = [_rows(R, H), _rows(R, SROW), _rows(R, DROW)]
        out_shape = [jax.ShapeDtypeStruct((N_PL, H), jnp.float32),
                     jax.ShapeDtypeStruct((N_PL, SROW), jnp.float32),
                     jax.ShapeDtypeStruct((N_PL, DROW), jnp.float32)]
    else:
        out_specs = _rows(R, H)
        out_shape = jax.ShapeDtypeStruct((N_PL, H), jnp.float32)
    return pl.pallas_call(
        functools.partial(_node_body, emit_next),
        grid=grid,
        in_specs=specs,
        out_specs=out_specs,
        out_shape=out_shape,
    )(*ins)


def _sc_gather2(E):
    per_w = E // NW
    n_chunks = per_w // CHUNK
    mesh = plsc.VectorSubcoreMesh(core_axis_name="c", subcore_axis_name="s")

    @functools.partial(
        pl.kernel,
        out_type=[jax.ShapeDtypeStruct((E, SROW), jnp.float32),
                  jax.ShapeDtypeStruct((E, DROW), jnp.float32)],
        mesh=mesh,
        scratch_types=[
            pltpu.VMEM((CHUNK,), jnp.int32),
            pltpu.VMEM((CHUNK,), jnp.int32),
            pltpu.VMEM((CHUNK, SROW), jnp.float32),
            pltpu.VMEM((CHUNK, DROW), jnp.float32),
            pltpu.SemaphoreType.DMA,
            pltpu.SemaphoreType.DMA,
        ],
    )
    def k(stab, sidx, dtab, didx, outs, outd, siv, div, srows, drows, sem1, sem2):
        c = lax.axis_index("c")
        s = lax.axis_index("s")
        wid = s * 2 + c
        base = wid * per_w

        def body(i, carry):
            off = base + i * CHUNK
            pltpu.sync_copy(sidx.at[pl.ds(off, CHUNK)], siv)
            pltpu.sync_copy(didx.at[pl.ds(off, CHUNK)], div)
            cp1 = pltpu.async_copy(stab.at[siv], srows, sem1)
            cp2 = pltpu.async_copy(dtab.at[div], drows, sem2)
            cp1.wait()
            cp2.wait()
            pltpu.sync_copy(srows, outs.at[pl.ds(off, CHUNK)])
            pltpu.sync_copy(drows, outd.at[pl.ds(off, CHUNK)])
            return carry

        lax.fori_loop(0, n_chunks, body, 0)

    return k


def _sc_scatter(E):
    per_c = E // 2
    per_w = per_c // 16
    n_chunks = per_w // CHUNK
    rows_per_tile = NSEG_PAD // 16
    mesh = plsc.VectorSubcoreMesh(core_axis_name="c", subcore_axis_name="s")

    @functools.partial(
        pl.kernel,
        out_type=jax.ShapeDtypeStruct((2 * NSEG_PAD, WROW), jnp.float32),
        mesh=mesh,
        scratch_types=[
            pltpu.VMEM((CHUNK,), jnp.int32),
            pltpu.VMEM((CHUNK, WROW), jnp.float32),
            pltpu.VMEM_SHARED((NSEG_PAD, WROW), jnp.float32),
        ],
    )
    def k(w_hbm, idx_hbm, zeros_hbm, out_hbm, idx_v, w_v, accum):
        c = lax.axis_index("c")
        s = lax.axis_index("s")
        r0 = s * rows_per_tile
        pltpu.sync_copy(zeros_hbm.at[pl.ds(r0, rows_per_tile)],
                        accum.at[pl.ds(r0, rows_per_tile)])
        plsc.subcore_barrier()
        base = c * per_c + s * per_w

        def body(i, carry):
            off = base + i * CHUNK
            pltpu.sync_copy(idx_hbm.at[pl.ds(off, CHUNK)], idx_v)
            pltpu.sync_copy(w_hbm.at[pl.ds(off, CHUNK)], w_v)
            pltpu.sync_copy(w_v, accum.at[idx_v], add=True)
            return carry

        lax.fori_loop(0, n_chunks, body, 0)
        plsc.subcore_barrier()
        pltpu.sync_copy(accum.at[pl.ds(r0, rows_per_tile)],
                        out_hbm.at[pl.ds(c * NSEG_PAD + r0, rows_per_tile)])

    return k


def _pad_i32(a, n):
    return jnp.pad(a.astype(jnp.int32), (0, n - a.shape[0]))


def kernel(params, pos_pt, orient_pt, magnitude_pt, pos_pl, orient_pl,
           type_pt, side_pt, type_pl, is_intersection_pl,
           edge_index_pt2pl, edge_index_pl2pl, type_pl2pl):
    p = params
    att1 = dict(p['pt2pl_layers'][0])
    att2 = dict(p['pl2pl_layers'][0])
    att1['embt'] = p['emb_type_pt']
    att1['embs'] = p['emb_side_pt']
    att1['emb4'] = p['emb_type_pl']
    att1['emb3'] = p['emb_int_pl']
    att1['Wq_k'] = att1['Wk']
    att1['Wq_v'] = att1['Wv']

    stab_pt = _pt_encode(pos_pt, orient_pt, magnitude_pt, type_pt, side_pt,
                         p['fe_x_pt'], att1)
    x_pl0, dtab0 = _pl_encode(type_pl, is_intersection_pl, pos_pl, orient_pl,
                              p['fe_x_pl'], att1)

    s1 = _pad_i32(edge_index_pt2pl[0], E1P)
    d1 = _pad_i32(edge_index_pt2pl[1], E1P)
    srows1, drows1 = _sc_gather2(E1P)(stab_pt, s1, dtab0, d1)

    t1 = jnp.zeros((E1P,), jnp.int32)
    emb5_zero = jnp.zeros((5, H), jnp.float32)
    w1 = _edge_call(E1P, E_PT2PL, srows1, drows1, t1,
                    p['fe_r_pt2pl'], emb5_zero, att1)

    zeros_acc = jnp.zeros((NSEG_PAD, WROW), jnp.float32)
    P1 = _sc_scatter(E1P)(w1, d1, zeros_acc)

    x_pl1, stab_pl, dtab1 = _node_call(P1, x_pl0, att1,
                                       nxt=(pos_pl, orient_pl, att2))

    s2 = _pad_i32(edge_index_pl2pl[0], E2P)
    d2 = _pad_i32(edge_index_pl2pl[1], E2P)
    srows2, drows2 = _sc_gather2(E2P)(stab_pl, s2, dtab1, d2)

    t2 = _pad_i32(type_pl2pl, E2P)
    w2 = _edge_call(E2P, E_PL2PL, srows2, drows2, t2,
                    p['fe_r_pl2pl'], p['emb_type_pl2pl'], att2)

    P2 = _sc_scatter(E2P)(w2, d2, zeros_acc)

    return _node_call(P2, x_pl1, att2)
